# baseline (speedup 1.0000x reference)
import functools

import jax
import jax.numpy as jnp
from jax import lax
from jax.experimental import pallas as pl
from jax.experimental.pallas import tpu as pltpu
from jax.experimental.pallas import tpu_sc as plsc

F = 26
V = 100000
D = 32
B = 16384

NC = 2
NS = 16
NW = NC * NS
BPW = B // NW
CH = 128
NCH = BPW // CH
L = 16


def _accumulate(acc, buf):
    def body(r, _):
        plsc.addupdate(acc.at[r, pl.ds(0, L)], buf[r, pl.ds(0, L)])
        plsc.addupdate(acc.at[r, pl.ds(L, L)], buf[r, pl.ds(L, L)])
        return 0

    lax.fori_loop(0, CH, body, 0, unroll=4)


@functools.partial(
    pl.kernel,
    mesh=plsc.VectorSubcoreMesh(core_axis_name="c", subcore_axis_name="s"),
    out_type=jax.ShapeDtypeStruct((B, D), jnp.float32),
    scratch_types=[
        pltpu.VMEM((F * NCH, CH), jnp.int32),
        pltpu.VMEM((CH, D), jnp.float32),
        pltpu.VMEM((CH, D), jnp.float32),
        pltpu.VMEM((CH, D), jnp.float32),
        pltpu.VMEM((CH, D), jnp.float32),
        pltpu.SemaphoreType.DMA,
        pltpu.SemaphoreType.DMA,
        pltpu.SemaphoreType.DMA,
        pltpu.SemaphoreType.DMA,
        pltpu.SemaphoreType.DMA,
    ],
    compiler_params=pltpu.CompilerParams(use_tc_tiling_on_sc=False),
)
def _emb_lookup_sum(tab, gid, out, idxv, acc0, acc1, g0, g1,
                    sema, sem0, sem1, semo0, semo1):
    wid = lax.axis_index("s") * NC + lax.axis_index("c")

    pltpu.sync_copy(gid.at[pl.ds(wid * F * NCH, F * NCH)], idxv)

    def gather(f, c, dst, sem):
        return pltpu.async_copy(tab.at[idxv.at[f * NCH + c]], dst, sem)

    def drain(dst, sem):
        pltpu.make_async_copy(tab.at[pl.ds(0, CH)], dst, sem).wait()

    accs = (acc0, acc1)
    semos = (semo0, semo1)
    out_cp = [None, None]
    for c in range(NCH):
        p = c % 2
        acc = accs[p]
        if out_cp[p] is not None:
            out_cp[p].wait()
        cp_a = gather(0, c, acc, sema)
        gather(1, c, g0, sem0)
        cp_a.wait()

        def pair(t, _):
            cp1 = gather(2 * t, c, g1, sem1)
            drain(g0, sem0)
            _accumulate(acc, g0)
            gather(2 * t + 1, c, g0, sem0)
            cp1.wait()
            _accumulate(acc, g1)
            return 0

        lax.fori_loop(1, 13, pair, 0)
        drain(g0, sem0)
        _accumulate(acc, g0)

        out_cp[p] = pltpu.async_copy(
            acc, out.at[pl.ds(wid * BPW + c * CH, CH)], semos[p])
    for p in range(2):
        if out_cp[p] is not None:
            out_cp[p].wait()


def kernel(x, tables):
    x = x.astype(jnp.int32)
    gid = x + (jnp.arange(F, dtype=jnp.int32) * V)[None, :]
    gid = (gid.reshape(NW, NCH, CH, F)
              .transpose(0, 3, 1, 2)
              .reshape(NW * F * NCH, CH))
    tab = tables.reshape(F * V, D)
    return _emb_lookup_sum(tab, gid)


# speedup vs baseline: 4.4047x; 4.4047x over previous
import functools

import jax
import jax.numpy as jnp
from jax import lax
from jax.experimental import pallas as pl
from jax.experimental.pallas import tpu as pltpu
from jax.experimental.pallas import tpu_sc as plsc

F = 26
V = 100000
D = 32
B = 16384

NC = 2
NS = 16
NW = NC * NS
L = 16
ICH = 4096
NICH = B // ICH


@functools.partial(
    pl.kernel,
    mesh=plsc.VectorSubcoreMesh(core_axis_name="c", subcore_axis_name="s"),
    out_type=jax.ShapeDtypeStruct((D, B), jnp.float32),
    scratch_types=[
        pltpu.VMEM((V,), jnp.float32),
        pltpu.VMEM((B,), jnp.float32),
        pltpu.VMEM((ICH,), jnp.int32),
        pltpu.VMEM((ICH,), jnp.int32),
        pltpu.SemaphoreType.DMA,
        pltpu.SemaphoreType.DMA,
        pltpu.SemaphoreType.DMA,
    ],
    compiler_params=pltpu.CompilerParams(needs_layout_passes=False),
)
def _emb_lookup_sum(tabfd, idxT, outT, row, acc, ib0, ib1, semr, semi0, semi1):
    d = lax.axis_index("s") * NC + lax.axis_index("c")

    def zero(i, _):
        acc[pl.ds(i * L, L)] = jnp.zeros((L,), jnp.float32)
        return 0

    lax.fori_loop(0, B // L, zero, 0, unroll=8)

    ibs = (ib0, ib1)
    semis = (semi0, semi1)

    def field(f, _):
        pltpu.async_copy(tabfd.at[f * D + d], row, semr).wait()
        pltpu.async_copy(idxT.at[f, pl.ds(0, ICH)], ib0, semi0)

        def chunk_body(c, ib, nxt):
            base = c * ICH

            def body(r, _):
                iv = ib[pl.ds(r * L, L)]
                g = plsc.load_gather(row, [iv])
                plsc.addupdate(acc.at[pl.ds(base + r * L, L)], g)
                return 0

            nxt()
            lax.fori_loop(0, ICH // L, body, 0, unroll=8)

        for c in range(NICH):
            p = c % 2
            q = 1 - p
            if c + 1 < NICH:
                def nxt(c=c, q=q):
                    pltpu.async_copy(
                        idxT.at[f, pl.ds((c + 1) * ICH, ICH)], ibs[q], semis[q])
            else:
                def nxt():
                    pass
            pltpu.make_async_copy(
                idxT.at[f, pl.ds(0, ICH)], ibs[p], semis[p]).wait()
            chunk_body(c, ibs[p], nxt)
        return 0

    lax.fori_loop(0, F, field, 0)
    pltpu.sync_copy(acc, outT.at[d])


def kernel(x, tables):
    x = x.astype(jnp.int32)
    xT = x.T
    tabfd = tables.transpose(0, 2, 1).reshape(F * D, V)
    outT = _emb_lookup_sum(tabfd, xT)
    return outT.T
